# Initial kernel scaffold; baseline (speedup 1.0000x reference)
import jax
import jax.numpy as jnp
from jax.experimental import pallas as pl


def kernel(x, edge_index, batch, W1, b1, W2, b2, fc1_W, fc1_b, fc2_W, fc2_b):
    raise NotImplementedError("write your pallas kernel here")


# speedup vs baseline: 12.5739x; 12.5739x over previous
import functools

import jax
import jax.numpy as jnp
from jax import lax
from jax.experimental import pallas as pl
from jax.experimental.pallas import tpu as pltpu
from jax.experimental.pallas import tpu_sc as plsc

N = 10000
E = 320000
D = 128
H = 128
A = 16
G = 64
MAX_ACTION = 1.0

NC = 2
NS = 16
NW = NC * NS
EPW = E // NW
C = 80
NCHUNK = EPW // C
NP = 10112
DEGW = 128
RPW = NP // NS
ZROWS = 158

_sc_mesh = plsc.VectorSubcoreMesh(core_axis_name="c", subcore_axis_name="s")


def _deg_body(dst_hbm, out_hbm, idx_v, ones_v, zbuf, deg_sh, sem):
    cid = lax.axis_index("c")
    sid = lax.axis_index("s")
    wid = sid * NC + cid

    onehot = jnp.where(lax.iota(jnp.int32, 16) == 0, 1.0, 0.0).astype(jnp.float32)
    zero16 = jnp.zeros((16,), jnp.float32)

    def init_rows(i, _):
        for j in range(DEGW // 16):
            ones_v[i, pl.ds(j * 16, 16)] = onehot if j == 0 else zero16
            zbuf[i, pl.ds(j * 16, 16)] = zero16
        return 0

    lax.fori_loop(0, C, init_rows, 0)

    for j in range(8):
        nj = C if j < 7 else RPW - 7 * C
        pltpu.sync_copy(zbuf.at[pl.ds(0, nj)],
                        deg_sh.at[pl.ds(sid * RPW + j * C, nj)])
    plsc.subcore_barrier()

    base = wid * EPW

    def step(k, _):
        pltpu.sync_copy(dst_hbm.at[pl.ds(base + k * C, C)], idx_v)
        pltpu.sync_copy(ones_v, deg_sh.at[idx_v], add=True)
        return 0

    lax.fori_loop(0, NCHUNK, step, 0)
    plsc.subcore_barrier()
    for j in range(8):
        nj = C if j < 7 else RPW - 7 * C
        pltpu.sync_copy(deg_sh.at[pl.ds(sid * RPW + j * C, nj)],
                        zbuf.at[pl.ds(0, nj)])
        pltpu.sync_copy(
            zbuf.at[pl.ds(0, nj)],
            out_hbm.at[pl.ds(cid * NP + sid * RPW + j * C, nj)])


_deg_kernel = functools.partial(
    pl.kernel,
    out_type=jax.ShapeDtypeStruct((NC * NP, DEGW), jnp.float32),
    mesh=_sc_mesh,
    scratch_types=[
        pltpu.VMEM((C,), jnp.int32),
        pltpu.VMEM((C, DEGW), jnp.float32),
        pltpu.VMEM((C, DEGW), jnp.float32),
        pltpu.VMEM_SHARED((NP, DEGW), jnp.float32),
        pltpu.SemaphoreType.DMA,
    ],
)(_deg_body)


def _scatter_body(g_hbm, src_hbm, dst_hbm, out_hbm,
                  idx_s, idx_d, rows, acc_sh, sem):
    cid = lax.axis_index("c")
    sid = lax.axis_index("s")
    wid = sid * NC + cid

    zero16 = jnp.zeros((16,), jnp.float32)

    def zrow(i, _):
        for j in range(H // 16):
            rows[i, pl.ds(j * 16, 16)] = zero16
        return 0

    lax.fori_loop(0, C, zrow, 0)
    for j in range(8):
        nj = C if j < 7 else RPW - 7 * C
        pltpu.sync_copy(rows.at[pl.ds(0, nj)],
                        acc_sh.at[pl.ds(sid * RPW + j * C, nj)])
    plsc.subcore_barrier()

    base = wid * EPW

    def step(k, _):
        pltpu.sync_copy(src_hbm.at[pl.ds(base + k * C, C)], idx_s)
        pltpu.sync_copy(dst_hbm.at[pl.ds(base + k * C, C)], idx_d)
        pltpu.async_copy(g_hbm.at[idx_s], rows, sem).wait()
        pltpu.sync_copy(rows, acc_sh.at[idx_d], add=True)
        return 0

    lax.fori_loop(0, NCHUNK, step, 0)
    plsc.subcore_barrier()
    for j in range(8):
        nj = C if j < 7 else RPW - 7 * C
        pltpu.sync_copy(acc_sh.at[pl.ds(sid * RPW + j * C, nj)],
                        rows.at[pl.ds(0, nj)])
        pltpu.sync_copy(
            rows.at[pl.ds(0, nj)],
            out_hbm.at[pl.ds(cid * NP + sid * RPW + j * C, nj)])


_scatter_kernel = functools.partial(
    pl.kernel,
    out_type=jax.ShapeDtypeStruct((NC * NP, H), jnp.float32),
    mesh=_sc_mesh,
    scratch_types=[
        pltpu.VMEM((C,), jnp.int32),
        pltpu.VMEM((C,), jnp.int32),
        pltpu.VMEM((C, H), jnp.float32),
        pltpu.VMEM_SHARED((NP, H), jnp.float32),
        pltpu.SemaphoreType.DMA,
    ],
)(_scatter_body)


def _layer1_body(deg_ref, x_ref, w_ref, g_ref, dinv_ref):
    deg = deg_ref[0:NP, 0:1] + deg_ref[NP:2 * NP, 0:1] + 1.0
    dinv = lax.rsqrt(deg)[0:N]
    dinv_ref[...] = dinv
    h = jnp.dot(x_ref[...], w_ref[...], preferred_element_type=jnp.float32)
    g_ref[...] = h * dinv


def _mid_body(acc_ref, g1_ref, dinv_ref, w2_ref, b1_ref, g2_ref):
    dinv = dinv_ref[...]
    agg = acc_ref[0:N, :] + acc_ref[NP:NP + N, :] + g1_ref[...]
    h1 = jnp.maximum(dinv * agg + b1_ref[...], 0.0)
    h2 = jnp.dot(h1, w2_ref[...], preferred_element_type=jnp.float32)
    g2_ref[...] = h2 * dinv


def _final_body(acc_ref, g2_ref, dinv_ref, b2_ref, batch_ref,
                fc1w_ref, fc1b_ref, fc2w_ref, fc2b_ref, out_ref):
    dinv = dinv_ref[...]
    agg = acc_ref[0:N, :] + acc_ref[NP:NP + N, :] + g2_ref[...]
    h2 = jnp.maximum(dinv * agg + b2_ref[...], 0.0)
    seg = lax.broadcasted_iota(jnp.int32, (1, G), 1)
    mask = (batch_ref[...] == seg).astype(jnp.float32)
    cnum = (((0,), (0,)), ((), ()))
    pooled = lax.dot_general(mask, h2, dimension_numbers=cnum,
                             preferred_element_type=jnp.float32)
    ones = jnp.ones((N, 1), jnp.float32)
    counts = lax.dot_general(mask, ones, dimension_numbers=cnum,
                             preferred_element_type=jnp.float32)
    pooled = pooled / jnp.maximum(counts, 1.0)
    z = jnp.maximum(
        jnp.dot(pooled, fc1w_ref[...], preferred_element_type=jnp.float32)
        + fc1b_ref[...], 0.0)
    o = jax.nn.sigmoid(
        jnp.dot(z, fc2w_ref[...], preferred_element_type=jnp.float32)
        + fc2b_ref[...])
    out_ref[...] = o * MAX_ACTION


def kernel(x, edge_index, batch, W1, b1, W2, b2, fc1_W, fc1_b, fc2_W, fc2_b):
    src = edge_index[0].astype(jnp.int32)
    dst = edge_index[1].astype(jnp.int32)
    batch2d = batch.astype(jnp.int32).reshape(N, 1)

    deg2 = _deg_kernel(dst)

    g1, dinv = pl.pallas_call(
        _layer1_body,
        out_shape=[
            jax.ShapeDtypeStruct((N, H), jnp.float32),
            jax.ShapeDtypeStruct((N, 1), jnp.float32),
        ],
    )(deg2, x, W1)

    acc1 = _scatter_kernel(g1, src, dst)

    g2 = pl.pallas_call(
        _mid_body,
        out_shape=jax.ShapeDtypeStruct((N, H), jnp.float32),
    )(acc1, g1, dinv, W2, b1.reshape(1, H))

    acc2 = _scatter_kernel(g2, src, dst)

    out = pl.pallas_call(
        _final_body,
        out_shape=jax.ShapeDtypeStruct((G, A), jnp.float32),
    )(acc2, g2, dinv, b2.reshape(1, H), batch2d,
      fc1_W, fc1_b.reshape(1, H), fc2_W, fc2_b.reshape(1, A))
    return out


# speedup vs baseline: 20.4450x; 1.6260x over previous
import functools

import jax
import jax.numpy as jnp
from jax import lax
from jax.experimental import pallas as pl
from jax.experimental.pallas import tpu as pltpu
from jax.experimental.pallas import tpu_sc as plsc

N = 10000
E = 320000
D = 128
H = 128
A = 16
G = 64
MAX_ACTION = 1.0

NC = 2
NS = 16
NW = NC * NS
EPW = E // NW
C = 80
NCHUNK = EPW // C
NP = 10112
DEGW = 128
RPW = NP // NS
ZROWS = 158

_sc_mesh = plsc.VectorSubcoreMesh(core_axis_name="c", subcore_axis_name="s")


def _deg_body(dst_hbm, out_hbm, idx_v, ones_v, zbuf, deg_sh, sem):
    cid = lax.axis_index("c")
    sid = lax.axis_index("s")
    wid = sid * NC + cid

    onehot = jnp.where(lax.iota(jnp.int32, 16) == 0, 1.0, 0.0).astype(jnp.float32)
    zero16 = jnp.zeros((16,), jnp.float32)

    def init_rows(i, _):
        for j in range(DEGW // 16):
            ones_v[i, pl.ds(j * 16, 16)] = onehot if j == 0 else zero16
            zbuf[i, pl.ds(j * 16, 16)] = zero16
        return 0

    lax.fori_loop(0, C, init_rows, 0)

    for j in range(8):
        nj = C if j < 7 else RPW - 7 * C
        pltpu.sync_copy(zbuf.at[pl.ds(0, nj)],
                        deg_sh.at[pl.ds(sid * RPW + j * C, nj)])
    plsc.subcore_barrier()

    base = wid * EPW

    def step(k, _):
        pltpu.sync_copy(dst_hbm.at[pl.ds(base + k * C, C)], idx_v)
        pltpu.sync_copy(ones_v, deg_sh.at[idx_v], add=True)
        return 0

    lax.fori_loop(0, NCHUNK, step, 0)
    plsc.subcore_barrier()
    for j in range(8):
        nj = C if j < 7 else RPW - 7 * C
        pltpu.sync_copy(deg_sh.at[pl.ds(sid * RPW + j * C, nj)],
                        zbuf.at[pl.ds(0, nj)])
        pltpu.sync_copy(
            zbuf.at[pl.ds(0, nj)],
            out_hbm.at[pl.ds(cid * NP + sid * RPW + j * C, nj)])


_deg_kernel = functools.partial(
    pl.kernel,
    out_type=jax.ShapeDtypeStruct((NC * NP, DEGW), jnp.float32),
    mesh=_sc_mesh,
    scratch_types=[
        pltpu.VMEM((C,), jnp.int32),
        pltpu.VMEM((C, DEGW), jnp.float32),
        pltpu.VMEM((C, DEGW), jnp.float32),
        pltpu.VMEM_SHARED((NP, DEGW), jnp.float32),
        pltpu.SemaphoreType.DMA,
    ],
)(_deg_body)


def _scatter_body(g_hbm, src2_hbm, dst_hbm, out_hbm,
                  idx_all, idx_da, idx_db, rows_a, rows_b, acc_sh,
                  sem_ga, sem_gb, sem_da, sem_db):
    cid = lax.axis_index("c")
    sid = lax.axis_index("s")
    wid = sid * NC + cid

    zero16 = jnp.zeros((16,), jnp.float32)

    def zrow(i, _):
        for j in range(H // 16):
            rows_a[i, pl.ds(j * 16, 16)] = zero16
        return 0

    lax.fori_loop(0, C, zrow, 0)
    for j in range(8):
        nj = C if j < 7 else RPW - 7 * C
        pltpu.sync_copy(rows_a.at[pl.ds(0, nj)],
                        acc_sh.at[pl.ds(sid * RPW + j * C, nj)])
    plsc.subcore_barrier()

    base = wid * EPW
    pltpu.sync_copy(src2_hbm.at[wid], idx_all)

    def gwait(buf, sem):
        pltpu.make_async_copy(g_hbm.at[pl.ds(0, C)], buf, sem).wait()

    def iwait(buf, sem):
        pltpu.make_async_copy(dst_hbm.at[pl.ds(0, C)], buf, sem).wait()

    pltpu.async_copy(dst_hbm.at[pl.ds(base, C)], idx_da, sem_da)
    pltpu.async_copy(dst_hbm.at[pl.ds(base + C, C)], idx_db, sem_db)
    pltpu.async_copy(g_hbm.at[idx_all.at[pl.ds(0, C)]], rows_a, sem_ga)

    def pair(t, _):
        ka = 2 * t
        gwait(rows_a, sem_ga)
        pltpu.async_copy(g_hbm.at[idx_all.at[pl.ds((ka + 1) * C, C)]],
                         rows_b, sem_gb)
        iwait(idx_da, sem_da)
        pltpu.sync_copy(rows_a, acc_sh.at[idx_da], add=True)
        pltpu.async_copy(dst_hbm.at[pl.ds(base + (ka + 2) * C, C)],
                         idx_da, sem_da)
        gwait(rows_b, sem_gb)
        pltpu.async_copy(g_hbm.at[idx_all.at[pl.ds((ka + 2) * C, C)]],
                         rows_a, sem_ga)
        iwait(idx_db, sem_db)
        pltpu.sync_copy(rows_b, acc_sh.at[idx_db], add=True)
        koff = jnp.minimum(ka + 3, NCHUNK - 1) * C
        pltpu.async_copy(dst_hbm.at[pl.ds(base + koff, C)], idx_db, sem_db)
        return 0

    lax.fori_loop(0, (NCHUNK - 1) // 2, pair, 0)
    gwait(rows_a, sem_ga)
    iwait(idx_da, sem_da)
    pltpu.sync_copy(rows_a, acc_sh.at[idx_da], add=True)
    iwait(idx_db, sem_db)
    plsc.subcore_barrier()
    for j in range(8):
        nj = C if j < 7 else RPW - 7 * C
        pltpu.sync_copy(acc_sh.at[pl.ds(sid * RPW + j * C, nj)],
                        rows_a.at[pl.ds(0, nj)])
        pltpu.sync_copy(
            rows_a.at[pl.ds(0, nj)],
            out_hbm.at[pl.ds(cid * NP + sid * RPW + j * C, nj)])


_scatter_kernel = functools.partial(
    pl.kernel,
    out_type=jax.ShapeDtypeStruct((NC * NP, H), jnp.float32),
    mesh=_sc_mesh,
    scratch_types=[
        pltpu.VMEM((EPW,), jnp.int32),
        pltpu.VMEM((C,), jnp.int32),
        pltpu.VMEM((C,), jnp.int32),
        pltpu.VMEM((C, H), jnp.float32),
        pltpu.VMEM((C, H), jnp.float32),
        pltpu.VMEM_SHARED((NP, H), jnp.float32),
        pltpu.SemaphoreType.DMA,
        pltpu.SemaphoreType.DMA,
        pltpu.SemaphoreType.DMA,
        pltpu.SemaphoreType.DMA,
    ],
)(_scatter_body)


def _layer1_body(deg_ref, x_ref, w_ref, g_ref, dinv_ref):
    deg = deg_ref[0:NP, 0:1] + deg_ref[NP:2 * NP, 0:1] + 1.0
    dinv = lax.rsqrt(deg)[0:N]
    dinv_ref[...] = dinv
    h = jnp.dot(x_ref[...], w_ref[...], preferred_element_type=jnp.float32)
    g_ref[...] = h * dinv


def _mid_body(acc_ref, g1_ref, dinv_ref, w2_ref, b1_ref, g2_ref):
    dinv = dinv_ref[...]
    agg = acc_ref[0:N, :] + acc_ref[NP:NP + N, :] + g1_ref[...]
    h1 = jnp.maximum(dinv * agg + b1_ref[...], 0.0)
    h2 = jnp.dot(h1, w2_ref[...], preferred_element_type=jnp.float32)
    g2_ref[...] = h2 * dinv


def _final_body(acc_ref, g2_ref, dinv_ref, b2_ref, batch_ref,
                fc1w_ref, fc1b_ref, fc2w_ref, fc2b_ref, out_ref):
    dinv = dinv_ref[...]
    agg = acc_ref[0:N, :] + acc_ref[NP:NP + N, :] + g2_ref[...]
    h2 = jnp.maximum(dinv * agg + b2_ref[...], 0.0)
    seg = lax.broadcasted_iota(jnp.int32, (1, G), 1)
    mask = (batch_ref[...] == seg).astype(jnp.float32)
    cnum = (((0,), (0,)), ((), ()))
    pooled = lax.dot_general(mask, h2, dimension_numbers=cnum,
                             preferred_element_type=jnp.float32)
    ones = jnp.ones((N, 1), jnp.float32)
    counts = lax.dot_general(mask, ones, dimension_numbers=cnum,
                             preferred_element_type=jnp.float32)
    pooled = pooled / jnp.maximum(counts, 1.0)
    z = jnp.maximum(
        jnp.dot(pooled, fc1w_ref[...], preferred_element_type=jnp.float32)
        + fc1b_ref[...], 0.0)
    o = jax.nn.sigmoid(
        jnp.dot(z, fc2w_ref[...], preferred_element_type=jnp.float32)
        + fc2b_ref[...])
    out_ref[...] = o * MAX_ACTION


def kernel(x, edge_index, batch, W1, b1, W2, b2, fc1_W, fc1_b, fc2_W, fc2_b):
    src = edge_index[0].astype(jnp.int32)
    src2 = src.reshape(NW, EPW)
    dst = edge_index[1].astype(jnp.int32)
    batch2d = batch.astype(jnp.int32).reshape(N, 1)

    deg2 = _deg_kernel(dst)

    g1, dinv = pl.pallas_call(
        _layer1_body,
        out_shape=[
            jax.ShapeDtypeStruct((N, H), jnp.float32),
            jax.ShapeDtypeStruct((N, 1), jnp.float32),
        ],
    )(deg2, x, W1)

    acc1 = _scatter_kernel(g1, src2, dst)

    g2 = pl.pallas_call(
        _mid_body,
        out_shape=jax.ShapeDtypeStruct((N, H), jnp.float32),
    )(acc1, g1, dinv, W2, b1.reshape(1, H))

    acc2 = _scatter_kernel(g2, src2, dst)

    out = pl.pallas_call(
        _final_body,
        out_shape=jax.ShapeDtypeStruct((G, A), jnp.float32),
    )(acc2, g2, dinv, b2.reshape(1, H), batch2d,
      fc1_W, fc1_b.reshape(1, H), fc2_W, fc2_b.reshape(1, A))
    return out


# speedup vs baseline: 22.0839x; 1.0802x over previous
import functools

import jax
import jax.numpy as jnp
from jax import lax
from jax.experimental import pallas as pl
from jax.experimental.pallas import tpu as pltpu
from jax.experimental.pallas import tpu_sc as plsc

N = 10000
E = 320000
D = 128
H = 128
A = 16
G = 64
MAX_ACTION = 1.0

NC = 2
NS = 16
NW = NC * NS
EPW = E // NW
C = 80
NCHUNK = EPW // C
NP = 10112
DEGW = 128
RPW = NP // NS
ZROWS = 158

_sc_mesh = plsc.VectorSubcoreMesh(core_axis_name="c", subcore_axis_name="s")


def _deg_body(dst_hbm, out_hbm, idx_a, idx_b, ones_v, zbuf, deg_sh,
              sem_da, sem_db, sem_sa, sem_sb):
    cid = lax.axis_index("c")
    sid = lax.axis_index("s")
    wid = sid * NC + cid

    onehot = jnp.where(lax.iota(jnp.int32, 16) == 0, 1.0, 0.0).astype(jnp.float32)
    zero16 = jnp.zeros((16,), jnp.float32)

    def init_rows(i, _):
        for j in range(DEGW // 16):
            ones_v[i, pl.ds(j * 16, 16)] = onehot if j == 0 else zero16
            zbuf[i, pl.ds(j * 16, 16)] = zero16
        return 0

    lax.fori_loop(0, C, init_rows, 0)

    for j in range(8):
        nj = C if j < 7 else RPW - 7 * C
        pltpu.sync_copy(zbuf.at[pl.ds(0, nj)],
                        deg_sh.at[pl.ds(sid * RPW + j * C, nj)])
    plsc.subcore_barrier()

    base = wid * EPW

    def iwait(buf, sem):
        pltpu.make_async_copy(dst_hbm.at[pl.ds(0, C)], buf, sem).wait()

    def swait(idx, sem):
        pltpu.make_async_copy(ones_v, deg_sh.at[idx], sem).wait()

    pltpu.async_copy(dst_hbm.at[pl.ds(base, C)], idx_a, sem_da)
    pltpu.async_copy(dst_hbm.at[pl.ds(base + C, C)], idx_b, sem_db)

    def pair(t, _):
        ka = 2 * t
        iwait(idx_a, sem_da)
        pltpu.async_copy(ones_v, deg_sh.at[idx_a], sem_sa, add=True)
        iwait(idx_b, sem_db)
        pltpu.async_copy(ones_v, deg_sh.at[idx_b], sem_sb, add=True)
        swait(idx_a, sem_sa)
        pltpu.async_copy(dst_hbm.at[pl.ds(base + (ka + 2) * C, C)],
                         idx_a, sem_da)
        swait(idx_b, sem_sb)
        koff = jnp.minimum(ka + 3, NCHUNK - 1) * C
        pltpu.async_copy(dst_hbm.at[pl.ds(base + koff, C)], idx_b, sem_db)
        return 0

    lax.fori_loop(0, (NCHUNK - 1) // 2, pair, 0)
    iwait(idx_a, sem_da)
    pltpu.async_copy(ones_v, deg_sh.at[idx_a], sem_sa, add=True)
    iwait(idx_b, sem_db)
    swait(idx_a, sem_sa)
    plsc.subcore_barrier()
    for j in range(8):
        nj = C if j < 7 else RPW - 7 * C
        pltpu.sync_copy(deg_sh.at[pl.ds(sid * RPW + j * C, nj)],
                        zbuf.at[pl.ds(0, nj)])
        pltpu.sync_copy(
            zbuf.at[pl.ds(0, nj)],
            out_hbm.at[pl.ds(cid * NP + sid * RPW + j * C, nj)])


_deg_kernel = functools.partial(
    pl.kernel,
    out_type=jax.ShapeDtypeStruct((NC * NP, DEGW), jnp.float32),
    mesh=_sc_mesh,
    scratch_types=[
        pltpu.VMEM((C,), jnp.int32),
        pltpu.VMEM((C,), jnp.int32),
        pltpu.VMEM((C, DEGW), jnp.float32),
        pltpu.VMEM((C, DEGW), jnp.float32),
        pltpu.VMEM_SHARED((NP, DEGW), jnp.float32),
        pltpu.SemaphoreType.DMA,
        pltpu.SemaphoreType.DMA,
        pltpu.SemaphoreType.DMA,
        pltpu.SemaphoreType.DMA,
    ],
)(_deg_body)


def _scatter_body(g_hbm, src2_hbm, dst_hbm, out_hbm,
                  idx_all, idx_da, idx_db, rows_a, rows_b, acc_sh,
                  sem_ga, sem_gb, sem_da, sem_db, sem_sa, sem_sb):
    cid = lax.axis_index("c")
    sid = lax.axis_index("s")
    wid = sid * NC + cid

    zero16 = jnp.zeros((16,), jnp.float32)

    def zrow(i, _):
        for j in range(H // 16):
            rows_a[i, pl.ds(j * 16, 16)] = zero16
        return 0

    lax.fori_loop(0, C, zrow, 0)
    for j in range(8):
        nj = C if j < 7 else RPW - 7 * C
        pltpu.sync_copy(rows_a.at[pl.ds(0, nj)],
                        acc_sh.at[pl.ds(sid * RPW + j * C, nj)])
    plsc.subcore_barrier()

    base = wid * EPW
    pltpu.sync_copy(src2_hbm.at[wid], idx_all)

    def gwait(buf, sem):
        pltpu.make_async_copy(g_hbm.at[pl.ds(0, C)], buf, sem).wait()

    def iwait(buf, sem):
        pltpu.make_async_copy(dst_hbm.at[pl.ds(0, C)], buf, sem).wait()

    pltpu.async_copy(dst_hbm.at[pl.ds(base, C)], idx_da, sem_da)
    pltpu.async_copy(dst_hbm.at[pl.ds(base + C, C)], idx_db, sem_db)
    pltpu.async_copy(g_hbm.at[idx_all.at[pl.ds(0, C)]], rows_a, sem_ga)
    pltpu.async_copy(g_hbm.at[idx_all.at[pl.ds(C, C)]], rows_b, sem_gb)

    def swait(rows, idx, sem):
        pltpu.make_async_copy(rows, acc_sh.at[idx], sem).wait()

    def pair(t, _):
        ka = 2 * t
        gwait(rows_a, sem_ga)
        iwait(idx_da, sem_da)
        pltpu.async_copy(rows_a, acc_sh.at[idx_da], sem_sa, add=True)
        gwait(rows_b, sem_gb)
        iwait(idx_db, sem_db)
        pltpu.async_copy(rows_b, acc_sh.at[idx_db], sem_sb, add=True)
        swait(rows_a, idx_da, sem_sa)
        pltpu.async_copy(g_hbm.at[idx_all.at[pl.ds((ka + 2) * C, C)]],
                         rows_a, sem_ga)
        pltpu.async_copy(dst_hbm.at[pl.ds(base + (ka + 2) * C, C)],
                         idx_da, sem_da)
        swait(rows_b, idx_db, sem_sb)
        koff = jnp.minimum(ka + 3, NCHUNK - 1) * C
        pltpu.async_copy(g_hbm.at[idx_all.at[pl.ds(koff, C)]],
                         rows_b, sem_gb)
        pltpu.async_copy(dst_hbm.at[pl.ds(base + koff, C)], idx_db, sem_db)
        return 0

    lax.fori_loop(0, (NCHUNK - 1) // 2, pair, 0)
    gwait(rows_a, sem_ga)
    iwait(idx_da, sem_da)
    pltpu.async_copy(rows_a, acc_sh.at[idx_da], sem_sa, add=True)
    gwait(rows_b, sem_gb)
    iwait(idx_db, sem_db)
    swait(rows_a, idx_da, sem_sa)
    plsc.subcore_barrier()
    for j in range(8):
        nj = C if j < 7 else RPW - 7 * C
        pltpu.sync_copy(acc_sh.at[pl.ds(sid * RPW + j * C, nj)],
                        rows_a.at[pl.ds(0, nj)])
        pltpu.sync_copy(
            rows_a.at[pl.ds(0, nj)],
            out_hbm.at[pl.ds(cid * NP + sid * RPW + j * C, nj)])


_scatter_kernel = functools.partial(
    pl.kernel,
    out_type=jax.ShapeDtypeStruct((NC * NP, H), jnp.float32),
    mesh=_sc_mesh,
    scratch_types=[
        pltpu.VMEM((EPW,), jnp.int32),
        pltpu.VMEM((C,), jnp.int32),
        pltpu.VMEM((C,), jnp.int32),
        pltpu.VMEM((C, H), jnp.float32),
        pltpu.VMEM((C, H), jnp.float32),
        pltpu.VMEM_SHARED((NP, H), jnp.float32),
        pltpu.SemaphoreType.DMA,
        pltpu.SemaphoreType.DMA,
        pltpu.SemaphoreType.DMA,
        pltpu.SemaphoreType.DMA,
        pltpu.SemaphoreType.DMA,
        pltpu.SemaphoreType.DMA,
    ],
)(_scatter_body)


def _layer1_body(deg_ref, x_ref, w_ref, g_ref, dinv_ref):
    deg = deg_ref[0:NP, 0:1] + deg_ref[NP:2 * NP, 0:1] + 1.0
    dinv = lax.rsqrt(deg)[0:N]
    dinv_ref[...] = dinv
    h = jnp.dot(x_ref[...], w_ref[...], preferred_element_type=jnp.float32)
    g_ref[...] = h * dinv


def _mid_body(acc_ref, g1_ref, dinv_ref, w2_ref, b1_ref, g2_ref):
    dinv = dinv_ref[...]
    agg = acc_ref[0:N, :] + acc_ref[NP:NP + N, :] + g1_ref[...]
    h1 = jnp.maximum(dinv * agg + b1_ref[...], 0.0)
    h2 = jnp.dot(h1, w2_ref[...], preferred_element_type=jnp.float32)
    g2_ref[...] = h2 * dinv


def _final_body(acc_ref, g2_ref, dinv_ref, b2_ref, batch_ref,
                fc1w_ref, fc1b_ref, fc2w_ref, fc2b_ref, out_ref):
    dinv = dinv_ref[...]
    agg = acc_ref[0:N, :] + acc_ref[NP:NP + N, :] + g2_ref[...]
    h2 = jnp.maximum(dinv * agg + b2_ref[...], 0.0)
    seg = lax.broadcasted_iota(jnp.int32, (1, G), 1)
    mask = (batch_ref[...] == seg).astype(jnp.float32)
    cnum = (((0,), (0,)), ((), ()))
    pooled = lax.dot_general(mask, h2, dimension_numbers=cnum,
                             preferred_element_type=jnp.float32)
    ones = jnp.ones((N, 1), jnp.float32)
    counts = lax.dot_general(mask, ones, dimension_numbers=cnum,
                             preferred_element_type=jnp.float32)
    pooled = pooled / jnp.maximum(counts, 1.0)
    z = jnp.maximum(
        jnp.dot(pooled, fc1w_ref[...], preferred_element_type=jnp.float32)
        + fc1b_ref[...], 0.0)
    o = jax.nn.sigmoid(
        jnp.dot(z, fc2w_ref[...], preferred_element_type=jnp.float32)
        + fc2b_ref[...])
    out_ref[...] = o * MAX_ACTION


def kernel(x, edge_index, batch, W1, b1, W2, b2, fc1_W, fc1_b, fc2_W, fc2_b):
    src = edge_index[0].astype(jnp.int32)
    src2 = src.reshape(NW, EPW)
    dst = edge_index[1].astype(jnp.int32)
    batch2d = batch.astype(jnp.int32).reshape(N, 1)

    deg2 = _deg_kernel(dst)

    g1, dinv = pl.pallas_call(
        _layer1_body,
        out_shape=[
            jax.ShapeDtypeStruct((N, H), jnp.float32),
            jax.ShapeDtypeStruct((N, 1), jnp.float32),
        ],
    )(deg2, x, W1)

    acc1 = _scatter_kernel(g1, src2, dst)

    g2 = pl.pallas_call(
        _mid_body,
        out_shape=jax.ShapeDtypeStruct((N, H), jnp.float32),
    )(acc1, g1, dinv, W2, b1.reshape(1, H))

    acc2 = _scatter_kernel(g2, src2, dst)

    out = pl.pallas_call(
        _final_body,
        out_shape=jax.ShapeDtypeStruct((G, A), jnp.float32),
    )(acc2, g2, dinv, b2.reshape(1, H), batch2d,
      fc1_W, fc1_b.reshape(1, H), fc2_W, fc2_b.reshape(1, A))
    return out
